# baseline (speedup 1.0000x reference)
import jax
import jax.numpy as jnp
from jax import lax
from jax.experimental import pallas as pl
from jax.experimental.pallas import tpu as pltpu
from jax.experimental.pallas import tpu_sc as plsc

N = 10000
E = 320000
D = 128
HID = 32

NUM_WORKERS = 32
CHUNK = 512
NCHUNKS = E // CHUNK
SUB = 128
NSUB = CHUNK // SUB
NGROUPS = CHUNK // 16
BASE_CHUNKS = NCHUNKS // NUM_WORKERS
NPAIRS = (BASE_CHUNKS + 2) // 2
CB = 8


def _tc_precompute(x_ref, wa_ref, wb_ref, b1_ref, p_ref, q_ref):
    xv = x_ref[...]
    p_ref[...] = jnp.dot(xv, wa_ref[...], preferred_element_type=jnp.float32)
    q_ref[...] = (jnp.dot(xv, wb_ref[...], preferred_element_type=jnp.float32)
                  + b1_ref[...])


def _sc_edge_mlp(p_hbm, q_hbm, ei_hbm, ef_hbm, w_hbm, out_hbm,
                 oidx_v, didx_v, rows_o, rows_d, ef_v, out_v, acc_v, w_v,
                 sems):
    wid = lax.axis_index("s") * 2 + lax.axis_index("c")
    pltpu.sync_copy(w_hbm, w_v)
    b2vec = jnp.full((16,), w_v[pl.ds(160, 16)][0], jnp.float32)

    def stage(cid, buf):
        pltpu.sync_copy(ei_hbm.at[0, cid], oidx_v.at[buf])
        pltpu.sync_copy(ei_hbm.at[1, cid], didx_v.at[buf])
        for j in range(NSUB):
            pltpu.async_copy(p_hbm.at[oidx_v.at[buf, j]],
                             rows_o.at[buf, pl.ds(j * SUB, SUB)], sems.at[buf])
            pltpu.async_copy(q_hbm.at[didx_v.at[buf, j]],
                             rows_d.at[buf, pl.ds(j * SUB, SUB)], sems.at[buf])
        pltpu.async_copy(ef_hbm.at[cid], ef_v.at[buf], sems.at[buf])

    def drain(buf):
        for j in range(NSUB):
            pltpu.make_async_copy(p_hbm.at[oidx_v.at[buf, j]],
                                  rows_o.at[buf, pl.ds(j * SUB, SUB)],
                                  sems.at[buf]).wait()
            pltpu.make_async_copy(q_hbm.at[didx_v.at[buf, j]],
                                  rows_d.at[buf, pl.ds(j * SUB, SUB)],
                                  sems.at[buf]).wait()
        pltpu.make_async_copy(ef_hbm.at[0], ef_v.at[buf], sems.at[buf]).wait()

    def compute(cid, buf):
        base = cid * CHUNK
        for cb in range(HID // CB):
            half = (cb * CB) // 16 * 16
            sub = cb * CB - half
            wrows = [w_v[pl.ds(j * HID + half, 16)] for j in range(4)]
            w2row = w_v[pl.ds(4 * HID + half, 16)]
            wsp = [[jnp.full((16,), wrows[j][sub + u], jnp.float32)
                    for u in range(CB)] for j in range(4)]
            w2sp = [jnp.full((16,), w2row[sub + u], jnp.float32)
                    for u in range(CB)]

            @plsc.parallel_loop(0, NGROUPS, unroll=2)
            def _(g, cb=cb, wsp=wsp, w2sp=w2sp):
                evec = g * 16 + lax.iota(jnp.int32, 16)
                ef = [plsc.load_gather(
                          ef_v.at[buf], [evec, jnp.full((16,), j, jnp.int32)])
                      for j in range(4)]
                part0 = b2vec if cb == 0 else acc_v[pl.ds(g * 16, 16)]
                part1 = jnp.zeros((16,), jnp.float32)
                parts = [part0, part1]
                for u in range(CB):
                    c = cb * CB + u
                    csplat = jnp.full((16,), c, jnp.int32)
                    h = (plsc.load_gather(rows_o.at[buf], [evec, csplat])
                         + plsc.load_gather(rows_d.at[buf], [evec, csplat]))
                    h = (h + ef[0] * wsp[0][u] + ef[1] * wsp[1][u]
                         + ef[2] * wsp[2][u] + ef[3] * wsp[3][u])
                    h = jnp.maximum(h, 0.01 * h)
                    parts[u % 2] = parts[u % 2] + h * w2sp[u]
                part = parts[0] + parts[1]
                if cb == HID // CB - 1:
                    out_v[pl.ds(g * 16, 16)] = jnp.maximum(part, 0.01 * part)
                else:
                    acc_v[pl.ds(g * 16, 16)] = part
        pltpu.sync_copy(out_v, out_hbm.at[pl.ds(base, CHUNK)])

    stage(wid, 0)

    def pair_body(k, carry):
        i1 = 2 * k + 1
        cid1 = wid + NUM_WORKERS * i1

        @pl.when(cid1 < NCHUNKS)
        def _():
            stage(cid1, 1)

        drain(0)
        compute(wid + NUM_WORKERS * 2 * k, 0)

        i2 = 2 * k + 2
        cid2 = wid + NUM_WORKERS * i2

        @pl.when(cid2 < NCHUNKS)
        def _():
            stage(cid2, 0)

        @pl.when(cid1 < NCHUNKS)
        def _():
            drain(1)
            compute(cid1, 1)

        return carry

    lax.fori_loop(0, NPAIRS, pair_body, 0)


def kernel(x, edge_index, edge_features, W1, b1, W2, b2):
    W1a = W1[:D]
    W1b = W1[D:2 * D]
    W1c = W1[2 * D:]
    p, q = pl.pallas_call(
        _tc_precompute,
        out_shape=[jax.ShapeDtypeStruct((N, HID), jnp.float32),
                   jax.ShapeDtypeStruct((N, HID), jnp.float32)],
    )(x, W1a, W1b, b1.reshape(1, HID))

    wpack = jnp.concatenate([W1c.reshape(-1), W2.reshape(-1), b2,
                             jnp.zeros((15,), jnp.float32)])

    ei4 = edge_index.reshape(2, NCHUNKS, NSUB, SUB)
    ef3 = edge_features.reshape(NCHUNKS, CHUNK, 4)

    mesh = plsc.VectorSubcoreMesh(core_axis_name="c", subcore_axis_name="s")
    run = pl.kernel(
        _sc_edge_mlp,
        mesh=mesh,
        compiler_params=pltpu.CompilerParams(
            needs_layout_passes=False, use_tc_tiling_on_sc=False),
        out_type=jax.ShapeDtypeStruct((E,), jnp.float32),
        scratch_types=[
            pltpu.VMEM((2, NSUB, SUB), jnp.int32),
            pltpu.VMEM((2, NSUB, SUB), jnp.int32),
            pltpu.VMEM((2, CHUNK, HID), jnp.float32),
            pltpu.VMEM((2, CHUNK, HID), jnp.float32),
            pltpu.VMEM((2, CHUNK, 4), jnp.float32),
            pltpu.VMEM((CHUNK,), jnp.float32),
            pltpu.VMEM((CHUNK,), jnp.float32),
            pltpu.VMEM((176,), jnp.float32),
            pltpu.SemaphoreType.DMA((2,)),
        ],
    )
    return run(p, q, ei4, ef3, wpack)


# speedup vs baseline: 1.0259x; 1.0259x over previous
import jax
import jax.numpy as jnp
from jax import lax
from jax.experimental import pallas as pl
from jax.experimental.pallas import tpu as pltpu
from jax.experimental.pallas import tpu_sc as plsc

N = 10000
E = 320000
D = 128
HID = 32

NUM_WORKERS = 32
CHUNK = 512
NCHUNKS = E // CHUNK
SUB = 128
NSUB = CHUNK // SUB
NGROUPS = CHUNK // 16
BASE_CHUNKS = NCHUNKS // NUM_WORKERS
NPAIRS = (BASE_CHUNKS + 2) // 2
CB = 4


def _tc_precompute(x_ref, wa_ref, wb_ref, b1_ref, p_ref, q_ref):
    xv = x_ref[...]
    p_ref[...] = jnp.dot(xv, wa_ref[...], preferred_element_type=jnp.float32)
    q_ref[...] = (jnp.dot(xv, wb_ref[...], preferred_element_type=jnp.float32)
                  + b1_ref[...])


def _sc_edge_mlp(p_hbm, q_hbm, ei_hbm, ef_hbm, w_hbm, out_hbm,
                 oidx_v, didx_v, rows_o, rows_d, ef_v, out_v, acc_v, w_v,
                 sems):
    wid = lax.axis_index("s") * 2 + lax.axis_index("c")
    pltpu.sync_copy(w_hbm, w_v)
    b2vec = jnp.full((16,), w_v[pl.ds(160, 16)][0], jnp.float32)

    def stage(cid, buf):
        pltpu.sync_copy(ei_hbm.at[0, cid], oidx_v.at[buf])
        pltpu.sync_copy(ei_hbm.at[1, cid], didx_v.at[buf])
        for j in range(NSUB):
            pltpu.async_copy(p_hbm.at[oidx_v.at[buf, j]],
                             rows_o.at[buf, pl.ds(j * SUB, SUB)], sems.at[buf])
            pltpu.async_copy(q_hbm.at[didx_v.at[buf, j]],
                             rows_d.at[buf, pl.ds(j * SUB, SUB)], sems.at[buf])
        pltpu.async_copy(ef_hbm.at[cid], ef_v.at[buf], sems.at[buf])

    def drain(buf):
        for j in range(NSUB):
            pltpu.make_async_copy(p_hbm.at[oidx_v.at[buf, j]],
                                  rows_o.at[buf, pl.ds(j * SUB, SUB)],
                                  sems.at[buf]).wait()
            pltpu.make_async_copy(q_hbm.at[didx_v.at[buf, j]],
                                  rows_d.at[buf, pl.ds(j * SUB, SUB)],
                                  sems.at[buf]).wait()
        pltpu.make_async_copy(ef_hbm.at[0], ef_v.at[buf], sems.at[buf]).wait()

    def compute(cid, buf):
        base = cid * CHUNK
        for cb in range(HID // CB):
            half = (cb * CB) // 16 * 16
            sub = cb * CB - half
            wrows = [w_v[pl.ds(j * HID + half, 16)] for j in range(4)]
            w2row = w_v[pl.ds(4 * HID + half, 16)]
            wsp = [[jnp.full((16,), wrows[j][sub + u], jnp.float32)
                    for u in range(CB)] for j in range(4)]
            w2sp = [jnp.full((16,), w2row[sub + u], jnp.float32)
                    for u in range(CB)]

            @plsc.parallel_loop(0, NGROUPS, unroll=2)
            def _(g, cb=cb, wsp=wsp, w2sp=w2sp):
                evec = g * 16 + lax.iota(jnp.int32, 16)
                ef = [plsc.load_gather(
                          ef_v.at[buf], [evec, jnp.full((16,), j, jnp.int32)])
                      for j in range(4)]
                part0 = b2vec if cb == 0 else acc_v[pl.ds(g * 16, 16)]
                part1 = jnp.zeros((16,), jnp.float32)
                parts = [part0, part1]
                for u in range(CB):
                    c = cb * CB + u
                    csplat = jnp.full((16,), c, jnp.int32)
                    h = (plsc.load_gather(rows_o.at[buf], [evec, csplat])
                         + plsc.load_gather(rows_d.at[buf], [evec, csplat]))
                    h = (h + ef[0] * wsp[0][u] + ef[1] * wsp[1][u]
                         + ef[2] * wsp[2][u] + ef[3] * wsp[3][u])
                    h = jnp.maximum(h, 0.01 * h)
                    parts[u % 2] = parts[u % 2] + h * w2sp[u]
                part = parts[0] + parts[1]
                if cb == HID // CB - 1:
                    out_v[pl.ds(g * 16, 16)] = jnp.maximum(part, 0.01 * part)
                else:
                    acc_v[pl.ds(g * 16, 16)] = part
        pltpu.sync_copy(out_v, out_hbm.at[pl.ds(base, CHUNK)])

    stage(wid, 0)

    def pair_body(k, carry):
        i1 = 2 * k + 1
        cid1 = wid + NUM_WORKERS * i1

        @pl.when(cid1 < NCHUNKS)
        def _():
            stage(cid1, 1)

        drain(0)
        compute(wid + NUM_WORKERS * 2 * k, 0)

        i2 = 2 * k + 2
        cid2 = wid + NUM_WORKERS * i2

        @pl.when(cid2 < NCHUNKS)
        def _():
            stage(cid2, 0)

        @pl.when(cid1 < NCHUNKS)
        def _():
            drain(1)
            compute(cid1, 1)

        return carry

    lax.fori_loop(0, NPAIRS, pair_body, 0)


def kernel(x, edge_index, edge_features, W1, b1, W2, b2):
    W1a = W1[:D]
    W1b = W1[D:2 * D]
    W1c = W1[2 * D:]
    p, q = pl.pallas_call(
        _tc_precompute,
        out_shape=[jax.ShapeDtypeStruct((N, HID), jnp.float32),
                   jax.ShapeDtypeStruct((N, HID), jnp.float32)],
    )(x, W1a, W1b, b1.reshape(1, HID))

    wpack = jnp.concatenate([W1c.reshape(-1), W2.reshape(-1), b2,
                             jnp.zeros((15,), jnp.float32)])

    ei4 = edge_index.reshape(2, NCHUNKS, NSUB, SUB)
    ef3 = edge_features.reshape(NCHUNKS, CHUNK, 4)

    mesh = plsc.VectorSubcoreMesh(core_axis_name="c", subcore_axis_name="s")
    run = pl.kernel(
        _sc_edge_mlp,
        mesh=mesh,
        compiler_params=pltpu.CompilerParams(
            needs_layout_passes=False, use_tc_tiling_on_sc=False),
        out_type=jax.ShapeDtypeStruct((E,), jnp.float32),
        scratch_types=[
            pltpu.VMEM((2, NSUB, SUB), jnp.int32),
            pltpu.VMEM((2, NSUB, SUB), jnp.int32),
            pltpu.VMEM((2, CHUNK, HID), jnp.float32),
            pltpu.VMEM((2, CHUNK, HID), jnp.float32),
            pltpu.VMEM((2, CHUNK, 4), jnp.float32),
            pltpu.VMEM((CHUNK,), jnp.float32),
            pltpu.VMEM((CHUNK,), jnp.float32),
            pltpu.VMEM((176,), jnp.float32),
            pltpu.SemaphoreType.DMA((2,)),
        ],
    )
    return run(p, q, ei4, ef3, wpack)


# speedup vs baseline: 1.5010x; 1.4631x over previous
import jax
import jax.numpy as jnp
from jax import lax
from jax.experimental import pallas as pl
from jax.experimental.pallas import tpu as pltpu
from jax.experimental.pallas import tpu_sc as plsc

N = 10000
E = 320000
D = 128
HID = 32

NUM_WORKERS = 32
CHUNK = 512
NCHUNKS = E // CHUNK
SUB = 128
NSUB = CHUNK // SUB
NGROUPS = CHUNK // 16
BASE_CHUNKS = NCHUNKS // NUM_WORKERS
NPAIRS = (BASE_CHUNKS + 2) // 2


def _tc_precompute(x_ref, wa_ref, wb_ref, b1_ref, p_ref, q_ref):
    xv = x_ref[...]
    p_ref[...] = jnp.dot(xv, wa_ref[...], preferred_element_type=jnp.float32)
    q_ref[...] = (jnp.dot(xv, wb_ref[...], preferred_element_type=jnp.float32)
                  + b1_ref[...])


def _sc_edge_mlp(p_hbm, q_hbm, ei_hbm, ef_hbm, w_hbm, out_hbm,
                 oidx_v, didx_v, rows_o, rows_d, ef_v, out_v, w_v, sems):
    wid = lax.axis_index("s") * 2 + lax.axis_index("c")
    pltpu.sync_copy(w_hbm, w_v)
    b2vec = jnp.full((16,), w_v[pl.ds(320, 16)][0], jnp.float32)

    def stage(cid, buf):
        pltpu.sync_copy(ei_hbm.at[0, cid], oidx_v.at[buf])
        pltpu.sync_copy(ei_hbm.at[1, cid], didx_v.at[buf])
        for j in range(NSUB):
            pltpu.async_copy(p_hbm.at[oidx_v.at[buf, j]],
                             rows_o.at[buf, pl.ds(j * SUB, SUB)], sems.at[buf])
            pltpu.async_copy(q_hbm.at[didx_v.at[buf, j]],
                             rows_d.at[buf, pl.ds(j * SUB, SUB)], sems.at[buf])
        pltpu.async_copy(ef_hbm.at[cid], ef_v.at[buf], sems.at[buf])

    def drain(buf):
        for j in range(NSUB):
            pltpu.make_async_copy(p_hbm.at[oidx_v.at[buf, j]],
                                  rows_o.at[buf, pl.ds(j * SUB, SUB)],
                                  sems.at[buf]).wait()
            pltpu.make_async_copy(q_hbm.at[didx_v.at[buf, j]],
                                  rows_d.at[buf, pl.ds(j * SUB, SUB)],
                                  sems.at[buf]).wait()
        pltpu.make_async_copy(ef_hbm.at[0], ef_v.at[buf], sems.at[buf]).wait()

    def compute(cid, buf):
        base = cid * CHUNK

        @plsc.parallel_loop(0, NGROUPS, unroll=2)
        def _(g):
            lane = lax.iota(jnp.int32, 16)
            evec = g * 16 + lane
            ef = [plsc.load_gather(
                      ef_v.at[buf], [evec, jnp.full((16,), j, jnp.int32)])
                  for j in range(4)]
            parts = [b2vec,
                     jnp.zeros((16,), jnp.float32),
                     jnp.zeros((16,), jnp.float32),
                     jnp.zeros((16,), jnp.float32)]
            for d in range(HID):
                cvec = (d + lane) & (HID - 1)
                h = (plsc.load_gather(rows_o.at[buf], [evec, cvec])
                     + plsc.load_gather(rows_d.at[buf], [evec, cvec]))
                h = (h + ef[0] * w_v[pl.ds(d, 16)]
                     + ef[1] * w_v[pl.ds(64 + d, 16)]
                     + ef[2] * w_v[pl.ds(128 + d, 16)]
                     + ef[3] * w_v[pl.ds(192 + d, 16)])
                h = jnp.maximum(h, 0.01 * h)
                parts[d % 4] = parts[d % 4] + h * w_v[pl.ds(256 + d, 16)]
            acc = (parts[0] + parts[1]) + (parts[2] + parts[3])
            out_v[pl.ds(g * 16, 16)] = jnp.maximum(acc, 0.01 * acc)

        pltpu.sync_copy(out_v, out_hbm.at[pl.ds(base, CHUNK)])

    stage(wid, 0)

    def pair_body(k, carry):
        i1 = 2 * k + 1
        cid1 = wid + NUM_WORKERS * i1

        @pl.when(cid1 < NCHUNKS)
        def _():
            stage(cid1, 1)

        drain(0)
        compute(wid + NUM_WORKERS * 2 * k, 0)

        i2 = 2 * k + 2
        cid2 = wid + NUM_WORKERS * i2

        @pl.when(cid2 < NCHUNKS)
        def _():
            stage(cid2, 0)

        @pl.when(cid1 < NCHUNKS)
        def _():
            drain(1)
            compute(cid1, 1)

        return carry

    lax.fori_loop(0, NPAIRS, pair_body, 0)


def kernel(x, edge_index, edge_features, W1, b1, W2, b2):
    W1a = W1[:D]
    W1b = W1[D:2 * D]
    W1c = W1[2 * D:]
    p, q = pl.pallas_call(
        _tc_precompute,
        out_shape=[jax.ShapeDtypeStruct((N, HID), jnp.float32),
                   jax.ShapeDtypeStruct((N, HID), jnp.float32)],
    )(x, W1a, W1b, b1.reshape(1, HID))

    w2f = W2.reshape(-1)
    wpack = jnp.concatenate(
        [jnp.concatenate([W1c[j], W1c[j]]) for j in range(4)]
        + [w2f, w2f, b2, jnp.zeros((15,), jnp.float32)])

    ei4 = edge_index.reshape(2, NCHUNKS, NSUB, SUB)
    ef3 = edge_features.reshape(NCHUNKS, CHUNK, 4)

    mesh = plsc.VectorSubcoreMesh(core_axis_name="c", subcore_axis_name="s")
    run = pl.kernel(
        _sc_edge_mlp,
        mesh=mesh,
        compiler_params=pltpu.CompilerParams(
            needs_layout_passes=False, use_tc_tiling_on_sc=False),
        out_type=jax.ShapeDtypeStruct((E,), jnp.float32),
        scratch_types=[
            pltpu.VMEM((2, NSUB, SUB), jnp.int32),
            pltpu.VMEM((2, NSUB, SUB), jnp.int32),
            pltpu.VMEM((2, CHUNK, HID), jnp.float32),
            pltpu.VMEM((2, CHUNK, HID), jnp.float32),
            pltpu.VMEM((2, CHUNK, 4), jnp.float32),
            pltpu.VMEM((CHUNK,), jnp.float32),
            pltpu.VMEM((336,), jnp.float32),
            pltpu.SemaphoreType.DMA((2,)),
        ],
    )
    return run(p, q, ei4, ef3, wpack)
